# Initial kernel scaffold; baseline (speedup 1.0000x reference)
import jax
import jax.numpy as jnp
from jax.experimental import pallas as pl


def kernel(x, edge_index, W_lin, b_lin, W_conv1, W_conv2, bn1_gamma, bn1_beta, W_lin1, b_lin1, bn2_gamma, bn2_beta, W_lin2, b_lin2):
    raise NotImplementedError("write your pallas kernel here")


# speedup vs baseline: 4.4382x; 4.4382x over previous
import functools

import jax
import jax.numpy as jnp
from jax import lax
from jax.experimental import pallas as pl
from jax.experimental.pallas import tpu as pltpu
from jax.experimental.pallas import tpu_sc as plsc

N = 10000
E = 320000
F = 128
C = 128
ALPHA = 0.1
THETA = 0.5
EPS = 1e-5

NPAD = 10112
NUM_TILES = 32
CHUNK = 128
CHUNKS_PER_TILE = 79
EPADDED = NUM_TILES * CHUNKS_PER_TILE * CHUNK
ROWS_PER_SUB_PAD = NPAD // 16


def _seg_body(h_pad, e3, zeros, out, src_idx, dst_idx, rows, acc, sem):
    c = lax.axis_index("c")
    s = lax.axis_index("s")
    wid = c * 16 + s
    r0 = s * ROWS_PER_SUB_PAD
    pltpu.sync_copy(zeros.at[pl.ds(r0, ROWS_PER_SUB_PAD)],
                    acc.at[pl.ds(r0, ROWS_PER_SUB_PAD)])
    pltpu.sync_copy(e3.at[0, wid], src_idx)
    pltpu.sync_copy(e3.at[1, wid], dst_idx)
    plsc.subcore_barrier()

    def body(j, carry):
        pltpu.async_copy(h_pad.at[src_idx.at[j]], rows, sem).wait()
        pltpu.sync_copy(rows, acc.at[dst_idx.at[j]], add=True)
        return carry

    lax.fori_loop(0, CHUNKS_PER_TILE, body, 0)
    plsc.subcore_barrier()
    pltpu.sync_copy(acc.at[pl.ds(r0, ROWS_PER_SUB_PAD)],
                    out.at[c, pl.ds(r0, ROWS_PER_SUB_PAD)])


_seg_partials = pl.kernel(
    _seg_body,
    mesh=plsc.VectorSubcoreMesh(core_axis_name="c", subcore_axis_name="s"),
    out_type=jax.ShapeDtypeStruct((2, NPAD, C), jnp.float32),
    scratch_types=[
        pltpu.VMEM((CHUNKS_PER_TILE, CHUNK), jnp.int32),
        pltpu.VMEM((CHUNKS_PER_TILE, CHUNK), jnp.int32),
        pltpu.VMEM((CHUNK, C), jnp.float32),
        pltpu.VMEM_SHARED((NPAD, C), jnp.float32),
        pltpu.SemaphoreType.DMA,
    ],
)


def _k1_body(x_ref, w_ref, b_ref, o_ref):
    x0 = jnp.dot(x_ref[...], w_ref[...], preferred_element_type=jnp.float32)
    x0 = jnp.maximum(x0 + b_ref[...], 0.0)
    o_ref[0:N, :] = x0
    o_ref[N:NPAD, :] = jnp.zeros((NPAD - N, C), jnp.float32)


def _layer_body(beta_l, p_ref, x0_ref, w_ref, g_ref, be_ref, o_ref):
    agg = (p_ref[0] + p_ref[1]) * (1.0 - ALPHA) + ALPHA * x0_ref[...]
    h = agg * (1.0 - beta_l) + jnp.dot(
        agg, w_ref[...], preferred_element_type=jnp.float32) * beta_l
    row = lax.broadcasted_iota(jnp.int32, (NPAD, 1), 0)
    valid = row < N
    m = jnp.sum(h, axis=0, keepdims=True) / N
    d = jnp.where(valid, h - m, 0.0)
    v = jnp.sum(d * d, axis=0, keepdims=True) / N
    hn = d * lax.rsqrt(v + EPS) * g_ref[...] + be_ref[...]
    hn = jnp.maximum(hn, 0.0)
    o_ref[...] = jnp.where(valid, hn, 0.0)


def _head_body(h_ref, w1_ref, b1_ref, g_ref, be_ref, w2_ref, b2_ref, o_ref):
    h = h_ref[0:N, :]
    z = jnp.dot(h, w1_ref[...], preferred_element_type=jnp.float32) + b1_ref[...]
    m = jnp.sum(z, axis=0, keepdims=True) / N
    d = z - m
    v = jnp.sum(d * d, axis=0, keepdims=True) / N
    zn = d * lax.rsqrt(v + EPS) * g_ref[...] + be_ref[...]
    o_ref[...] = jnp.sum(zn * w2_ref[...], axis=1, keepdims=True) + b2_ref[...]


def kernel(x, edge_index, W_lin, b_lin, W_conv1, W_conv2, bn1_gamma, bn1_beta,
           W_lin1, b_lin1, bn2_gamma, bn2_beta, W_lin2, b_lin2):
    import numpy as np
    pad = jnp.full((2, EPADDED - E), N, dtype=jnp.int32)
    e3 = jnp.concatenate([edge_index, pad], axis=1).reshape(
        2, NUM_TILES, CHUNKS_PER_TILE, CHUNK)
    zeros = jnp.zeros((NPAD, C), jnp.float32)

    x0p = pl.pallas_call(
        _k1_body,
        out_shape=jax.ShapeDtypeStruct((NPAD, C), jnp.float32),
    )(x, W_lin, b_lin.reshape(1, C))

    h = x0p
    for layer, W in enumerate([W_conv1, W_conv2], start=1):
        beta_l = float(np.log(THETA / layer + 1.0))
        parts = _seg_partials(h, e3, zeros)
        h = pl.pallas_call(
            functools.partial(_layer_body, beta_l),
            out_shape=jax.ShapeDtypeStruct((NPAD, C), jnp.float32),
        )(parts, x0p, W, bn1_gamma.reshape(1, C), bn1_beta.reshape(1, C))

    out = pl.pallas_call(
        _head_body,
        out_shape=jax.ShapeDtypeStruct((N, 1), jnp.float32),
    )(h, W_lin1, b_lin1.reshape(1, 16), bn2_gamma.reshape(1, 16),
      bn2_beta.reshape(1, 16), W_lin2.reshape(1, 16), b_lin2.reshape(1, 1))
    return out
